# baseline (speedup 1.0000x reference)
import functools

import jax
import jax.numpy as jnp
from jax import lax
from jax.experimental import pallas as pl
from jax.experimental.pallas import tpu as pltpu
from jax.experimental.pallas import tpu_sc as plsc

_NSLOT = 3
_DEPTH = 2
_SUBG = 2


def _gather_fn(n_total, n_vocab, d_model, n_cores, n_subcores, gchunk,
               n_chunks):
    n_workers = n_cores * n_subcores
    n_per_w = n_total // n_workers
    chunk = gchunk * _SUBG

    mesh = plsc.VectorSubcoreMesh(core_axis_name="c", subcore_axis_name="s")

    @functools.partial(
        pl.kernel,
        out_type=jax.ShapeDtypeStruct((n_total, d_model), jnp.float32),
        mesh=mesh,
        scratch_types=[
            pltpu.VMEM_SHARED((n_vocab, d_model), jnp.float32),
            pltpu.VMEM((n_per_w,), jnp.int32),
            pltpu.VMEM((_NSLOT, chunk, d_model), jnp.float32),
            pltpu.SemaphoreType.DMA((_NSLOT,)),
            pltpu.SemaphoreType.DMA((_NSLOT,)),
        ],
    )
    def run(idx_hbm, table_hbm, out_hbm, table_s, idx_v, rows_v, sem_g,
            sem_s):
        sid = lax.axis_index("s")
        wid = sid * n_cores + lax.axis_index("c")
        base = wid * n_per_w

        @pl.when(sid == 0)
        def _():
            pltpu.sync_copy(table_hbm, table_s)

        pltpu.sync_copy(idx_hbm.at[pl.ds(base, n_per_w)], idx_v)
        plsc.subcore_barrier()

        def sub_gathers(i, slot):
            return [
                pltpu.make_async_copy(
                    table_s.at[idx_v.at[pl.ds(i * chunk + g * gchunk,
                                              gchunk)]],
                    rows_v.at[slot].at[pl.ds(g * gchunk, gchunk)],
                    sem_g.at[slot],
                )
                for g in range(_SUBG)
            ]

        def gather_start(i, slot):
            for d in sub_gathers(i, slot):
                d.start()

        def gather_wait(i, slot):
            for d in sub_gathers(i, slot):
                d.wait()

        def store(i, slot):
            return pltpu.make_async_copy(
                rows_v.at[slot],
                out_hbm.at[pl.ds(base + i * chunk, chunk)],
                sem_s.at[slot],
            )

        for b in range(_DEPTH):
            gather_start(b, b)

        gather_wait(0, 0)
        store(0, 0).start()
        gather_start(_DEPTH, _DEPTH % _NSLOT)

        def body(i, carry):
            slot = lax.rem(i, _NSLOT)
            nslot = lax.rem(i + _DEPTH, _NSLOT)
            gather_wait(i, slot)
            store(i, slot).start()
            store(i - 1, nslot).wait()
            gather_start(i + _DEPTH, nslot)
            return carry

        lax.fori_loop(1, n_chunks - _DEPTH, body, 0)

        for k in range(n_chunks - _DEPTH, n_chunks):
            slot = lax.rem(k, _NSLOT)
            gather_wait(k, slot)
            store(k, slot).start()

        for k in range(n_chunks - _NSLOT, n_chunks):
            store(k, lax.rem(k, _NSLOT)).wait()

    return run


def kernel(positions, pe):
    b, s = positions.shape
    v, d = pe.shape
    n_total = b * s
    idx_flat = positions.reshape(n_total).astype(jnp.int32)

    info = plsc.get_sparse_core_info()
    n_cores, n_subcores = info.num_cores, info.num_subcores
    n_workers = n_cores * n_subcores
    n_per_w = n_total // n_workers
    gchunk = 128
    n_chunks = n_per_w // (gchunk * _SUBG)

    out = _gather_fn(n_total, v, d, n_cores, n_subcores, gchunk, n_chunks)(
        idx_flat, pe
    )
    return out.reshape(b, s, d)


# speedup vs baseline: 1.0659x; 1.0659x over previous
import functools

import jax
import jax.numpy as jnp
from jax import lax
from jax.experimental import pallas as pl
from jax.experimental.pallas import tpu as pltpu
from jax.experimental.pallas import tpu_sc as plsc

_NSLOT = 5
_DEPTH = 4


def _gather_fn(n_total, n_vocab, d_model, n_cores, n_subcores, chunk,
               n_chunks):
    n_workers = n_cores * n_subcores
    n_per_w = n_total // n_workers

    mesh = plsc.VectorSubcoreMesh(core_axis_name="c", subcore_axis_name="s")

    @functools.partial(
        pl.kernel,
        out_type=jax.ShapeDtypeStruct((n_total, d_model), jnp.float32),
        mesh=mesh,
        scratch_types=[
            pltpu.VMEM_SHARED((n_vocab, d_model), jnp.float32),
            pltpu.VMEM((n_per_w,), jnp.int32),
            pltpu.VMEM((_NSLOT, chunk, d_model), jnp.float32),
            pltpu.SemaphoreType.DMA((_NSLOT,)),
            pltpu.SemaphoreType.DMA((_NSLOT,)),
        ],
    )
    def run(idx_hbm, table_hbm, out_hbm, table_s, idx_v, rows_v, sem_g,
            sem_s):
        sid = lax.axis_index("s")
        wid = sid * n_cores + lax.axis_index("c")
        base = wid * n_per_w

        @pl.when(sid == 0)
        def _():
            pltpu.sync_copy(table_hbm, table_s)

        pltpu.sync_copy(idx_hbm.at[pl.ds(base, n_per_w)], idx_v)
        plsc.subcore_barrier()

        def gather(i, slot):
            return pltpu.make_async_copy(
                table_s.at[idx_v.at[pl.ds(i * chunk, chunk)]],
                rows_v.at[slot],
                sem_g.at[slot],
            )

        def store(i, slot):
            return pltpu.make_async_copy(
                rows_v.at[slot],
                out_hbm.at[pl.ds(base + i * chunk, chunk)],
                sem_s.at[slot],
            )

        for b in range(_DEPTH):
            gather(b, b).start()

        for b in range(_NSLOT):
            gather(b, b).wait()
            store(b, b).start()
            nslot = (b + _DEPTH) % _NSLOT
            if b >= 1:
                store(b - 1, nslot).wait()
            gather(b + _DEPTH, nslot).start()

        def body(g, carry):
            for b in range(_NSLOT):
                i = g * _NSLOT + b
                nslot = (b + _DEPTH) % _NSLOT
                gather(i, b).wait()
                store(i, b).start()
                store(i - 1, nslot).wait()
                gather(i + _DEPTH, nslot).start()
            return carry

        lax.fori_loop(1, n_chunks // _NSLOT - 1, body, 0)

        g_last = n_chunks // _NSLOT - 1
        for b in range(_NSLOT):
            i = g_last * _NSLOT + b
            nslot = (b + _DEPTH) % _NSLOT
            gather(i, b).wait()
            store(i, b).start()
            if i + _DEPTH < n_chunks:
                store(i - 1, nslot).wait()
                gather(i + _DEPTH, nslot).start()

        for b in range(_NSLOT):
            store(g_last * _NSLOT + b, b).wait()

    return run


def kernel(positions, pe):
    b, s = positions.shape
    v, d = pe.shape
    n_total = b * s
    idx_flat = positions.reshape(n_total).astype(jnp.int32)

    info = plsc.get_sparse_core_info()
    n_cores, n_subcores = info.num_cores, info.num_subcores
    n_workers = n_cores * n_subcores
    n_per_w = n_total // n_workers
    chunk = 128
    n_chunks = n_per_w // chunk

    out = _gather_fn(n_total, v, d, n_cores, n_subcores, chunk, n_chunks)(
        idx_flat, pe
    )
    return out.reshape(b, s, d)
